# baseline (speedup 1.0000x reference)
import functools

import jax
import jax.numpy as jnp
from jax import lax
from jax.experimental import pallas as pl
from jax.experimental.pallas import tpu as pltpu
from jax.experimental.pallas import tpu_sc as plsc

T, D, E, K, F, FS = 4096, 1024, 8, 2, 512, 1024
TS = 512
TM = 512
_NRAW = T * K + E * TM
NT = -(-_NRAW // TM)
NPAD = NT * TM
NW = 32
NEG = -1e30
DH = D // 2


def _pack_bf16(v):
    vb = v.astype(jnp.bfloat16)
    lo = jax.lax.bitcast_convert_type(vb[:, :DH], jnp.uint16).astype(jnp.uint32)
    hi = jax.lax.bitcast_convert_type(vb[:, DH:], jnp.uint16).astype(jnp.uint32)
    return jax.lax.bitcast_convert_type(lo | (hi << 16), jnp.int32)


def _unpack_bf16(w):
    u = jax.lax.bitcast_convert_type(w, jnp.uint32)
    lo = jax.lax.bitcast_convert_type((u & 0xFFFF).astype(jnp.uint16),
                                      jnp.bfloat16)
    hi = jax.lax.bitcast_convert_type((u >> 16).astype(jnp.uint16),
                                      jnp.bfloat16)
    return jnp.concatenate([lo, hi], axis=1)


def _router_body(x_ref, rw_ref,
                 xp_ref, e0_ref, e1_ref, r0_ref, r1_ref,
                 w0_ref, w1_ref, cnt_ref, carry_ref):
    i = pl.program_id(0)

    @pl.when(i == 0)
    def _():
        carry_ref[...] = jnp.zeros_like(carry_ref)

    x = x_ref[...]
    logits = jnp.dot(x, rw_ref[...], preferred_element_type=jnp.float32)
    ii = lax.broadcasted_iota(jnp.int32, (TS, E), 1)
    m0 = jnp.max(logits, axis=1, keepdims=True)
    e0 = jnp.min(jnp.where(logits == m0, ii, E), axis=1, keepdims=True)
    l2 = jnp.where(ii == e0, NEG, logits)
    m1 = jnp.max(l2, axis=1, keepdims=True)
    e1 = jnp.min(jnp.where(l2 == m1, ii, E), axis=1, keepdims=True)
    w0 = jax.nn.sigmoid(m0 - m1)

    oh0 = (ii == e0).astype(jnp.float32)
    oh1 = (ii == e1).astype(jnp.float32)
    oh = oh0 + oh1
    cum = oh
    k = 1
    while k < TS:
        cum = cum + jnp.concatenate(
            [jnp.zeros((k, E), jnp.float32), cum[:-k, :]], axis=0)
        k *= 2
    cum = cum - oh + carry_ref[0:1, :]
    r0 = jnp.sum(cum * oh0, axis=1, keepdims=True)
    r1 = jnp.sum(cum * oh1, axis=1, keepdims=True)
    carry_new = carry_ref[0:1, :] + jnp.sum(oh, axis=0, keepdims=True)
    carry_ref[...] = jnp.broadcast_to(carry_new, carry_ref.shape)
    cnt_ref[...] = jnp.broadcast_to(carry_new, cnt_ref.shape).astype(jnp.int32)

    e0_ref[...] = e0
    e1_ref[...] = e1
    r0_ref[...] = r0.astype(jnp.int32)
    r1_ref[...] = r1.astype(jnp.int32)
    w0_ref[...] = w0
    w1_ref[...] = 1.0 - w0

    xp_ref[...] = _pack_bf16(x)


def _router(x, rw):
    nb = T // TS
    meta_i = jax.ShapeDtypeStruct((T, 1), jnp.int32)
    meta_f = jax.ShapeDtypeStruct((T, 1), jnp.float32)
    col = lambda: pl.BlockSpec((TS, 1), lambda i: (i, 0))
    return pl.pallas_call(
        _router_body,
        grid=(nb,),
        in_specs=[
            pl.BlockSpec((TS, D), lambda i: (i, 0)),
            pl.BlockSpec((D, E), lambda i: (0, 0)),
        ],
        out_specs=[
            pl.BlockSpec((TS, DH), lambda i: (i, 0)),
            col(), col(), col(), col(), col(), col(),
            pl.BlockSpec((8, E), lambda i: (0, 0)),
        ],
        out_shape=[
            jax.ShapeDtypeStruct((T, DH), jnp.int32),
            meta_i, meta_i, meta_i, meta_i, meta_f, meta_f,
            jax.ShapeDtypeStruct((8, E), jnp.int32),
        ],
        scratch_shapes=[pltpu.VMEM((8, E), jnp.float32)],
    )(x, rw)


def _shared_body(x_ref, sg_ref, su_ref, sd_ref, shared_ref):
    xb = x_ref[...]
    sg = jnp.dot(xb, sg_ref[...], preferred_element_type=jnp.float32)
    su = jnp.dot(xb, su_ref[...], preferred_element_type=jnp.float32)
    h = sg * jax.nn.sigmoid(sg) * su
    shared_ref[...] = jnp.dot(h, sd_ref[...], preferred_element_type=jnp.float32)


def _shared_mlp(x, sg, su, sd):
    nb = T // TS
    return pl.pallas_call(
        _shared_body,
        grid=(nb,),
        in_specs=[
            pl.BlockSpec((TS, D), lambda i: (i, 0)),
            pl.BlockSpec((D, FS), lambda i: (0, 0)),
            pl.BlockSpec((D, FS), lambda i: (0, 0)),
            pl.BlockSpec((FS, D), lambda i: (0, 0)),
        ],
        out_specs=pl.BlockSpec((TS, D), lambda i: (i, 0)),
        out_shape=jax.ShapeDtypeStruct((T, D), jnp.float32),
    )(x, sg, su, sd)


def _meta_body(e0_ref, e1_ref, r0_ref, r1_ref, cnt_ref,
               p0_ref, p1_ref, texp_ref):
    cf = cnt_ref[...].astype(jnp.float32)
    i8r = lax.broadcasted_iota(jnp.int32, (E, E), 0)
    i8c = lax.broadcasted_iota(jnp.int32, (E, E), 1)
    eye = (i8r == i8c).astype(jnp.float32)
    le = (i8c <= i8r).astype(jnp.float32)
    c_col = jnp.sum(cf * eye, axis=1, keepdims=True)
    tiles_col = jnp.ceil(c_col / TM)
    cum_col = jnp.dot(le, tiles_col,
                      precision=jax.lax.Precision.HIGHEST)
    off_col = (cum_col - tiles_col) * TM
    off_row = jnp.sum(eye * off_col, axis=0, keepdims=True)

    ii = lax.broadcasted_iota(jnp.int32, (T, E), 1)
    oh0 = ii == e0_ref[...]
    oh1 = ii == e1_ref[...]
    zero = jnp.zeros((), jnp.float32)
    p0 = jnp.sum(jnp.where(oh0, off_row, zero), axis=1, keepdims=True)
    p1 = jnp.sum(jnp.where(oh1, off_row, zero), axis=1, keepdims=True)
    p0_ref[...] = p0.astype(jnp.int32) + r0_ref[...]
    p1_ref[...] = p1.astype(jnp.int32) + r1_ref[...]

    it = lax.broadcasted_iota(jnp.int32, (E, NT), 1).astype(jnp.float32)
    ge = (it >= cum_col).astype(jnp.int32)
    texp = jnp.minimum(jnp.sum(ge, axis=0, keepdims=True), E - 1)
    texp_ref[...] = jnp.broadcast_to(texp, (8, NT))


def _route_meta2(e0, e1, r0, r1, cnt):
    meta_i = jax.ShapeDtypeStruct((T, 1), jnp.int32)
    full = lambda: pl.BlockSpec((T, 1), lambda: (0, 0))
    p0, p1, texp = pl.pallas_call(
        _meta_body,
        in_specs=[full(), full(), full(), full(),
                  pl.BlockSpec((8, E), lambda: (0, 0))],
        out_specs=[full(), full(), pl.BlockSpec((8, NT), lambda: (0, 0))],
        out_shape=[meta_i, meta_i, jax.ShapeDtypeStruct((8, NT), jnp.int32)],
    )(e0, e1, r0, r1, cnt)
    return texp[0], p0[:, 0], p1[:, 0]


def _sc_scatter_rows(src, pos3, ch):
    t, d = src.shape
    per = t // NW
    nch = per // ch
    mesh = plsc.VectorSubcoreMesh(core_axis_name="c", subcore_axis_name="s")

    @functools.partial(
        pl.kernel, mesh=mesh,
        out_type=jax.ShapeDtypeStruct((NPAD, d), src.dtype),
        scratch_types=[
            pltpu.VMEM((nch, ch), jnp.int32),
            pltpu.VMEM((nch, ch), jnp.int32),
            [pltpu.VMEM((ch, d), src.dtype) for _ in range(2)],
            [pltpu.SemaphoreType.DMA for _ in range(2)],
            [pltpu.SemaphoreType.DMA for _ in range(2)],
            [pltpu.SemaphoreType.DMA for _ in range(2)],
        ],
    )
    def k(src_hbm, pos3_hbm, out_hbm, idx0_v, idx1_v, bufs, lsems, s0sems, s1sems):
        wid = lax.axis_index("s") * 2 + lax.axis_index("c")
        base = wid * per
        pltpu.sync_copy(pos3_hbm.at[0, pl.ds(wid * nch, nch)], idx0_v)
        pltpu.sync_copy(pos3_hbm.at[1, pl.ds(wid * nch, nch)], idx1_v)
        lops = [None] * nch
        sops = {}

        def load(j):
            b = j % 2
            lops[j] = pltpu.async_copy(
                src_hbm.at[pl.ds(base + j * ch, ch)], bufs[b], lsems[b])

        load(0)
        for j in range(nch):
            if j + 1 < nch:
                if j - 1 >= 0:
                    sops[(j - 1, 0)].wait()
                    sops[(j - 1, 1)].wait()
                load(j + 1)
            lops[j].wait()
            b = j % 2
            sops[(j, 0)] = pltpu.async_copy(
                bufs[b], out_hbm.at[idx0_v.at[j]], s0sems[b])
            sops[(j, 1)] = pltpu.async_copy(
                bufs[b], out_hbm.at[idx1_v.at[j]], s1sems[b])
        for j in range(max(0, nch - 2), nch):
            sops[(j, 0)].wait()
            sops[(j, 1)].wait()

    return k(src, pos3)


def _sc_gather_rows(src, idx, ch):
    m = idx.shape[0]
    d = src.shape[1]
    per = m // NW
    nch = per // ch
    nbuf = 2
    mesh = plsc.VectorSubcoreMesh(core_axis_name="c", subcore_axis_name="s")

    @functools.partial(
        pl.kernel, mesh=mesh,
        out_type=jax.ShapeDtypeStruct((m, d), src.dtype),
        scratch_types=[
            pltpu.VMEM((per,), jnp.int32),
            [pltpu.VMEM((ch, d), src.dtype) for _ in range(nbuf)],
            [pltpu.SemaphoreType.DMA for _ in range(nbuf)],
            [pltpu.SemaphoreType.DMA for _ in range(nbuf)],
        ],
    )
    def k(src_hbm, idx_hbm, out_hbm, idx_v, bufs, gsems, wsems):
        wid = lax.axis_index("s") * 2 + lax.axis_index("c")
        base = wid * per
        pltpu.sync_copy(idx_hbm.at[pl.ds(base, per)], idx_v)
        gops = [None] * nch
        wops = [None] * nch

        def start_gather(c):
            b = c % nbuf
            gops[c] = pltpu.async_copy(
                src_hbm.at[idx_v.at[pl.ds(c * ch, ch)]], bufs[b], gsems[b])

        start_gather(0)
        for c in range(nch):
            if c + 1 < nch:
                if c + 1 >= nbuf:
                    wops[c + 1 - nbuf].wait()
                start_gather(c + 1)
            gops[c].wait()
            b = c % nbuf
            wops[c] = pltpu.async_copy(
                bufs[b], out_hbm.at[pl.ds(base + c * ch, ch)], wsems[b])
        for c in range(max(0, nch - nbuf), nch):
            wops[c].wait()

    return k(src, idx)


def _grouped_body(texp_s, xg_ref, wg_ref, wu_ref, wd_ref, y_ref):
    xg = _unpack_bf16(xg_ref[...])
    hg = jnp.dot(xg, wg_ref[0], preferred_element_type=jnp.float32)
    hu = jnp.dot(xg, wu_ref[0], preferred_element_type=jnp.float32)
    h = (hg * jax.nn.sigmoid(hg) * hu).astype(jnp.bfloat16)
    y = jnp.dot(h, wd_ref[0], preferred_element_type=jnp.float32)
    y_ref[...] = _pack_bf16(y)


def _grouped_mlp(texp, xg, wg, wu, wd):
    grid_spec = pltpu.PrefetchScalarGridSpec(
        num_scalar_prefetch=1,
        grid=(NT,),
        in_specs=[
            pl.BlockSpec((TM, DH), lambda i, s: (i, 0)),
            pl.BlockSpec((1, D, F), lambda i, s: (s[i], 0, 0)),
            pl.BlockSpec((1, D, F), lambda i, s: (s[i], 0, 0)),
            pl.BlockSpec((1, F, D), lambda i, s: (s[i], 0, 0)),
        ],
        out_specs=pl.BlockSpec((TM, DH), lambda i, s: (i, 0)),
    )
    return pl.pallas_call(
        _grouped_body, grid_spec=grid_spec,
        out_shape=jax.ShapeDtypeStruct((NPAD, DH), jnp.int32),
    )(texp, xg, wg, wu, wd)


def _combine_body(ya_ref, yb_ref, sh_ref, w0_ref, w1_ref, o_ref):
    ya = _unpack_bf16(ya_ref[...]).astype(jnp.float32)
    yb = _unpack_bf16(yb_ref[...]).astype(jnp.float32)
    o_ref[...] = w0_ref[...] * ya + w1_ref[...] * yb + sh_ref[...]


def _combine(yg, shared_out, w0, w1):
    nb = T // TS
    return pl.pallas_call(
        _combine_body,
        grid=(nb,),
        in_specs=[
            pl.BlockSpec((TS, DH), lambda i: (i, 0)),
            pl.BlockSpec((TS, DH), lambda i, _nb=nb: (i + _nb, 0)),
            pl.BlockSpec((TS, D), lambda i: (i, 0)),
            pl.BlockSpec((TS, 1), lambda i: (i, 0)),
            pl.BlockSpec((TS, 1), lambda i: (i, 0)),
        ],
        out_specs=pl.BlockSpec((TS, D), lambda i: (i, 0)),
        out_shape=jax.ShapeDtypeStruct((T, D), jnp.float32),
    )(yg, yg, shared_out, w0, w1)


def kernel(hidden_states, router_w, w_gate, w_up, w_down,
           shared_gate, shared_up, shared_down):
    x = hidden_states
    (xp, e0, e1, r0, r1, w0, w1, cnt) = _router(x, router_w)
    texp, pos0, pos1 = _route_meta2(e0, e1, r0, r1, cnt)
    shared_out = _shared_mlp(x, shared_gate, shared_up, shared_down)
    ch = 32
    pos3 = jnp.stack([pos0, pos1]).reshape(2, T // ch, ch)
    bf = jnp.bfloat16
    xg = _sc_scatter_rows(xp, pos3, ch)
    y = _grouped_mlp(texp, xg, w_gate.astype(bf), w_up.astype(bf),
                     w_down.astype(bf))
    pos01 = jnp.concatenate([pos0, pos1])
    yg = _sc_gather_rows(y, pos01, 64)
    return _combine(yg, shared_out, w0, w1)


# speedup vs baseline: 1.0943x; 1.0943x over previous
import functools

import jax
import jax.numpy as jnp
from jax import lax
from jax.experimental import pallas as pl
from jax.experimental.pallas import tpu as pltpu
from jax.experimental.pallas import tpu_sc as plsc

T, D, E, K, F, FS = 4096, 1024, 8, 2, 512, 1024
TS = 1024
TM = 512
_NRAW = T * K + E * TM
NT = -(-_NRAW // TM)
NPAD = NT * TM
NW = 32
NEG = -1e30
DH = D // 2


def _pack_bf16(v):
    vb = v.astype(jnp.bfloat16)
    lo = jax.lax.bitcast_convert_type(vb[:, :DH], jnp.uint16).astype(jnp.uint32)
    hi = jax.lax.bitcast_convert_type(vb[:, DH:], jnp.uint16).astype(jnp.uint32)
    return jax.lax.bitcast_convert_type(lo | (hi << 16), jnp.int32)


def _unpack_bf16(w):
    u = jax.lax.bitcast_convert_type(w, jnp.uint32)
    lo = jax.lax.bitcast_convert_type((u & 0xFFFF).astype(jnp.uint16),
                                      jnp.bfloat16)
    hi = jax.lax.bitcast_convert_type((u >> 16).astype(jnp.uint16),
                                      jnp.bfloat16)
    return jnp.concatenate([lo, hi], axis=1)


def _router_body(x_ref, rw_ref,
                 xp_ref, e0_ref, e1_ref, r0_ref, r1_ref,
                 w0_ref, w1_ref, cnt_ref, carry_ref):
    i = pl.program_id(0)

    @pl.when(i == 0)
    def _():
        carry_ref[...] = jnp.zeros_like(carry_ref)

    x = x_ref[...]
    logits = jnp.dot(x, rw_ref[...], preferred_element_type=jnp.float32)
    ii = lax.broadcasted_iota(jnp.int32, (TS, E), 1)
    m0 = jnp.max(logits, axis=1, keepdims=True)
    e0 = jnp.min(jnp.where(logits == m0, ii, E), axis=1, keepdims=True)
    l2 = jnp.where(ii == e0, NEG, logits)
    m1 = jnp.max(l2, axis=1, keepdims=True)
    e1 = jnp.min(jnp.where(l2 == m1, ii, E), axis=1, keepdims=True)
    w0 = jax.nn.sigmoid(m0 - m1)

    oh0 = (ii == e0).astype(jnp.float32)
    oh1 = (ii == e1).astype(jnp.float32)
    oh = oh0 + oh1
    cum = oh
    k = 1
    while k < TS:
        cum = cum + jnp.concatenate(
            [jnp.zeros((k, E), jnp.float32), cum[:-k, :]], axis=0)
        k *= 2
    cum = cum - oh + carry_ref[0:1, :]
    r0 = jnp.sum(cum * oh0, axis=1, keepdims=True)
    r1 = jnp.sum(cum * oh1, axis=1, keepdims=True)
    carry_new = carry_ref[0:1, :] + jnp.sum(oh, axis=0, keepdims=True)
    carry_ref[...] = jnp.broadcast_to(carry_new, carry_ref.shape)
    cnt_ref[...] = jnp.broadcast_to(carry_new, cnt_ref.shape).astype(jnp.int32)

    e0_ref[...] = e0
    e1_ref[...] = e1
    r0_ref[...] = r0.astype(jnp.int32)
    r1_ref[...] = r1.astype(jnp.int32)
    w0_ref[...] = w0
    w1_ref[...] = 1.0 - w0

    xp_ref[...] = _pack_bf16(x)


def _router(x, rw):
    nb = T // TS
    meta_i = jax.ShapeDtypeStruct((T, 1), jnp.int32)
    meta_f = jax.ShapeDtypeStruct((T, 1), jnp.float32)
    col = lambda: pl.BlockSpec((TS, 1), lambda i: (i, 0))
    return pl.pallas_call(
        _router_body,
        grid=(nb,),
        in_specs=[
            pl.BlockSpec((TS, D), lambda i: (i, 0)),
            pl.BlockSpec((D, E), lambda i: (0, 0)),
        ],
        out_specs=[
            pl.BlockSpec((TS, DH), lambda i: (i, 0)),
            col(), col(), col(), col(), col(), col(),
            pl.BlockSpec((8, E), lambda i: (0, 0)),
        ],
        out_shape=[
            jax.ShapeDtypeStruct((T, DH), jnp.int32),
            meta_i, meta_i, meta_i, meta_i, meta_f, meta_f,
            jax.ShapeDtypeStruct((8, E), jnp.int32),
        ],
        scratch_shapes=[pltpu.VMEM((8, E), jnp.float32)],
    )(x, rw)


def _shared_body(x_ref, sg_ref, su_ref, sd_ref, shared_ref):
    xb = x_ref[...]
    sg = jnp.dot(xb, sg_ref[...], preferred_element_type=jnp.float32)
    su = jnp.dot(xb, su_ref[...], preferred_element_type=jnp.float32)
    h = sg * jax.nn.sigmoid(sg) * su
    shared_ref[...] = jnp.dot(h, sd_ref[...], preferred_element_type=jnp.float32)


def _shared_mlp(x, sg, su, sd):
    nb = T // TS
    return pl.pallas_call(
        _shared_body,
        grid=(nb,),
        in_specs=[
            pl.BlockSpec((TS, D), lambda i: (i, 0)),
            pl.BlockSpec((D, FS), lambda i: (0, 0)),
            pl.BlockSpec((D, FS), lambda i: (0, 0)),
            pl.BlockSpec((FS, D), lambda i: (0, 0)),
        ],
        out_specs=pl.BlockSpec((TS, D), lambda i: (i, 0)),
        out_shape=jax.ShapeDtypeStruct((T, D), jnp.float32),
    )(x, sg, su, sd)


def _meta_body(e0_ref, e1_ref, r0_ref, r1_ref, cnt_ref,
               p0_ref, p1_ref, texp_ref):
    cf = cnt_ref[...].astype(jnp.float32)
    i8r = lax.broadcasted_iota(jnp.int32, (E, E), 0)
    i8c = lax.broadcasted_iota(jnp.int32, (E, E), 1)
    eye = (i8r == i8c).astype(jnp.float32)
    le = (i8c <= i8r).astype(jnp.float32)
    c_col = jnp.sum(cf * eye, axis=1, keepdims=True)
    tiles_col = jnp.ceil(c_col / TM)
    cum_col = jnp.dot(le, tiles_col,
                      precision=jax.lax.Precision.HIGHEST)
    off_col = (cum_col - tiles_col) * TM
    off_row = jnp.sum(eye * off_col, axis=0, keepdims=True)

    ii = lax.broadcasted_iota(jnp.int32, (T, E), 1)
    oh0 = ii == e0_ref[...]
    oh1 = ii == e1_ref[...]
    zero = jnp.zeros((), jnp.float32)
    p0 = jnp.sum(jnp.where(oh0, off_row, zero), axis=1, keepdims=True)
    p1 = jnp.sum(jnp.where(oh1, off_row, zero), axis=1, keepdims=True)
    p0_ref[...] = p0.astype(jnp.int32) + r0_ref[...]
    p1_ref[...] = p1.astype(jnp.int32) + r1_ref[...]

    it = lax.broadcasted_iota(jnp.int32, (E, NT), 1).astype(jnp.float32)
    ge = (it >= cum_col).astype(jnp.int32)
    texp = jnp.minimum(jnp.sum(ge, axis=0, keepdims=True), E - 1)
    texp_ref[...] = jnp.broadcast_to(texp, (8, NT))


def _route_meta2(e0, e1, r0, r1, cnt):
    meta_i = jax.ShapeDtypeStruct((T, 1), jnp.int32)
    full = lambda: pl.BlockSpec((T, 1), lambda: (0, 0))
    p0, p1, texp = pl.pallas_call(
        _meta_body,
        in_specs=[full(), full(), full(), full(),
                  pl.BlockSpec((8, E), lambda: (0, 0))],
        out_specs=[full(), full(), pl.BlockSpec((8, NT), lambda: (0, 0))],
        out_shape=[meta_i, meta_i, jax.ShapeDtypeStruct((8, NT), jnp.int32)],
    )(e0, e1, r0, r1, cnt)
    return texp[0], p0[:, 0], p1[:, 0]


def _sc_scatter_rows(src, pos3, ch):
    t, d = src.shape
    per = t // NW
    nch = per // ch
    mesh = plsc.VectorSubcoreMesh(core_axis_name="c", subcore_axis_name="s")

    @functools.partial(
        pl.kernel, mesh=mesh,
        out_type=jax.ShapeDtypeStruct((NPAD, d), src.dtype),
        scratch_types=[
            pltpu.VMEM((nch, ch), jnp.int32),
            pltpu.VMEM((nch, ch), jnp.int32),
            [pltpu.VMEM((ch, d), src.dtype) for _ in range(2)],
            [pltpu.SemaphoreType.DMA for _ in range(2)],
            [pltpu.SemaphoreType.DMA for _ in range(2)],
            [pltpu.SemaphoreType.DMA for _ in range(2)],
        ],
    )
    def k(src_hbm, pos3_hbm, out_hbm, idx0_v, idx1_v, bufs, lsems, s0sems, s1sems):
        wid = lax.axis_index("s") * 2 + lax.axis_index("c")
        base = wid * per
        pltpu.sync_copy(pos3_hbm.at[0, pl.ds(wid * nch, nch)], idx0_v)
        pltpu.sync_copy(pos3_hbm.at[1, pl.ds(wid * nch, nch)], idx1_v)
        lops = [None] * nch
        sops = {}

        def load(j):
            b = j % 2
            lops[j] = pltpu.async_copy(
                src_hbm.at[pl.ds(base + j * ch, ch)], bufs[b], lsems[b])

        load(0)
        for j in range(nch):
            if j + 1 < nch:
                if j - 1 >= 0:
                    sops[(j - 1, 0)].wait()
                    sops[(j - 1, 1)].wait()
                load(j + 1)
            lops[j].wait()
            b = j % 2
            sops[(j, 0)] = pltpu.async_copy(
                bufs[b], out_hbm.at[idx0_v.at[j]], s0sems[b])
            sops[(j, 1)] = pltpu.async_copy(
                bufs[b], out_hbm.at[idx1_v.at[j]], s1sems[b])
        for j in range(max(0, nch - 2), nch):
            sops[(j, 0)].wait()
            sops[(j, 1)].wait()

    return k(src, pos3)


def _sc_gather_rows(src, idx, ch):
    m = idx.shape[0]
    d = src.shape[1]
    per = m // NW
    nch = per // ch
    nbuf = 2
    mesh = plsc.VectorSubcoreMesh(core_axis_name="c", subcore_axis_name="s")

    @functools.partial(
        pl.kernel, mesh=mesh,
        out_type=jax.ShapeDtypeStruct((m, d), src.dtype),
        scratch_types=[
            pltpu.VMEM((per,), jnp.int32),
            [pltpu.VMEM((ch, d), src.dtype) for _ in range(nbuf)],
            [pltpu.SemaphoreType.DMA for _ in range(nbuf)],
            [pltpu.SemaphoreType.DMA for _ in range(nbuf)],
        ],
    )
    def k(src_hbm, idx_hbm, out_hbm, idx_v, bufs, gsems, wsems):
        wid = lax.axis_index("s") * 2 + lax.axis_index("c")
        base = wid * per
        pltpu.sync_copy(idx_hbm.at[pl.ds(base, per)], idx_v)
        gops = [None] * nch
        wops = [None] * nch

        def start_gather(c):
            b = c % nbuf
            gops[c] = pltpu.async_copy(
                src_hbm.at[idx_v.at[pl.ds(c * ch, ch)]], bufs[b], gsems[b])

        start_gather(0)
        for c in range(nch):
            if c + 1 < nch:
                if c + 1 >= nbuf:
                    wops[c + 1 - nbuf].wait()
                start_gather(c + 1)
            gops[c].wait()
            b = c % nbuf
            wops[c] = pltpu.async_copy(
                bufs[b], out_hbm.at[pl.ds(base + c * ch, ch)], wsems[b])
        for c in range(max(0, nch - nbuf), nch):
            wops[c].wait()

    return k(src, idx)


def _grouped_body(texp_s, xg_ref, wg_ref, wu_ref, wd_ref, y_ref):
    xg = _unpack_bf16(xg_ref[...]).astype(jnp.float32)
    hg = jnp.dot(xg, wg_ref[0], preferred_element_type=jnp.float32)
    hu = jnp.dot(xg, wu_ref[0], preferred_element_type=jnp.float32)
    h = hg * jax.nn.sigmoid(hg) * hu
    y = jnp.dot(h, wd_ref[0], preferred_element_type=jnp.float32)
    y_ref[...] = _pack_bf16(y)


def _grouped_mlp(texp, xg, wg, wu, wd):
    grid_spec = pltpu.PrefetchScalarGridSpec(
        num_scalar_prefetch=1,
        grid=(NT,),
        in_specs=[
            pl.BlockSpec((TM, DH), lambda i, s: (i, 0)),
            pl.BlockSpec((1, D, F), lambda i, s: (s[i], 0, 0)),
            pl.BlockSpec((1, D, F), lambda i, s: (s[i], 0, 0)),
            pl.BlockSpec((1, F, D), lambda i, s: (s[i], 0, 0)),
        ],
        out_specs=pl.BlockSpec((TM, DH), lambda i, s: (i, 0)),
    )
    return pl.pallas_call(
        _grouped_body, grid_spec=grid_spec,
        out_shape=jax.ShapeDtypeStruct((NPAD, DH), jnp.int32),
    )(texp, xg, wg, wu, wd)


def _combine_body(ya_ref, yb_ref, sh_ref, w0_ref, w1_ref, o_ref):
    ya = _unpack_bf16(ya_ref[...]).astype(jnp.float32)
    yb = _unpack_bf16(yb_ref[...]).astype(jnp.float32)
    o_ref[...] = w0_ref[...] * ya + w1_ref[...] * yb + sh_ref[...]


def _combine(yg, shared_out, w0, w1):
    nb = T // TS
    return pl.pallas_call(
        _combine_body,
        grid=(nb,),
        in_specs=[
            pl.BlockSpec((TS, DH), lambda i: (i, 0)),
            pl.BlockSpec((TS, DH), lambda i, _nb=nb: (i + _nb, 0)),
            pl.BlockSpec((TS, D), lambda i: (i, 0)),
            pl.BlockSpec((TS, 1), lambda i: (i, 0)),
            pl.BlockSpec((TS, 1), lambda i: (i, 0)),
        ],
        out_specs=pl.BlockSpec((TS, D), lambda i: (i, 0)),
        out_shape=jax.ShapeDtypeStruct((T, D), jnp.float32),
    )(yg, yg, shared_out, w0, w1)


def kernel(hidden_states, router_w, w_gate, w_up, w_down,
           shared_gate, shared_up, shared_down):
    x = hidden_states
    (xp, e0, e1, r0, r1, w0, w1, cnt) = _router(x, router_w)
    texp, pos0, pos1 = _route_meta2(e0, e1, r0, r1, cnt)
    shared_out = _shared_mlp(x, shared_gate, shared_up, shared_down)
    ch = 32
    pos3 = jnp.stack([pos0, pos1]).reshape(2, T // ch, ch)
    xg = _sc_scatter_rows(xp, pos3, ch)
    y = _grouped_mlp(texp, xg, w_gate, w_up, w_down)
    pos01 = jnp.concatenate([pos0, pos1])
    yg = _sc_gather_rows(y, pos01, 64)
    return _combine(yg, shared_out, w0, w1)


# speedup vs baseline: 1.1215x; 1.0248x over previous
import functools

import jax
import jax.numpy as jnp
from jax import lax
from jax.experimental import pallas as pl
from jax.experimental.pallas import tpu as pltpu
from jax.experimental.pallas import tpu_sc as plsc

T, D, E, K, F, FS = 4096, 1024, 8, 2, 512, 1024
TS = 1024
TM = 512
_NRAW = T * K + E * TM
NT = -(-_NRAW // TM)
NPAD = NT * TM
NW = 32
NEG = -1e30
DH = D // 2


def _pack_bf16(v):
    vb = v.astype(jnp.bfloat16)
    lo = jax.lax.bitcast_convert_type(vb[:, :DH], jnp.uint16).astype(jnp.uint32)
    hi = jax.lax.bitcast_convert_type(vb[:, DH:], jnp.uint16).astype(jnp.uint32)
    return jax.lax.bitcast_convert_type(lo | (hi << 16), jnp.int32)


def _unpack_bf16(w):
    u = jax.lax.bitcast_convert_type(w, jnp.uint32)
    lo = jax.lax.bitcast_convert_type((u & 0xFFFF).astype(jnp.uint16),
                                      jnp.bfloat16)
    hi = jax.lax.bitcast_convert_type((u >> 16).astype(jnp.uint16),
                                      jnp.bfloat16)
    return jnp.concatenate([lo, hi], axis=1)


def _router_body(x_ref, rw_ref,
                 xp_ref, w0_ref, w1_ref, p0_ref, p1_ref, texp_ref,
                 carry_ref, es0, es1, rs0, rs1):
    i = pl.program_id(0)
    nb = T // TS

    @pl.when(i == 0)
    def _():
        carry_ref[...] = jnp.zeros_like(carry_ref)

    @pl.when(i < nb)
    def _():
        x = x_ref[...]
        logits = jnp.dot(x, rw_ref[...], preferred_element_type=jnp.float32)
        ii = lax.broadcasted_iota(jnp.int32, (TS, E), 1)
        m0 = jnp.max(logits, axis=1, keepdims=True)
        e0 = jnp.min(jnp.where(logits == m0, ii, E), axis=1, keepdims=True)
        l2 = jnp.where(ii == e0, NEG, logits)
        m1 = jnp.max(l2, axis=1, keepdims=True)
        e1 = jnp.min(jnp.where(l2 == m1, ii, E), axis=1, keepdims=True)
        w0 = jax.nn.sigmoid(m0 - m1)

        oh0 = (ii == e0).astype(jnp.float32)
        oh1 = (ii == e1).astype(jnp.float32)
        oh = oh0 + oh1
        cum = oh
        k = 1
        while k < TS:
            cum = cum + jnp.concatenate(
                [jnp.zeros((k, E), jnp.float32), cum[:-k, :]], axis=0)
            k *= 2
        cum = cum - oh + carry_ref[0:1, :]
        r0 = jnp.sum(cum * oh0, axis=1, keepdims=True)
        r1 = jnp.sum(cum * oh1, axis=1, keepdims=True)
        carry_new = carry_ref[0:1, :] + jnp.sum(oh, axis=0, keepdims=True)
        carry_ref[...] = jnp.broadcast_to(carry_new, carry_ref.shape)

        base = i * TS
        es0[pl.ds(base, TS), :] = e0
        es1[pl.ds(base, TS), :] = e1
        rs0[pl.ds(base, TS), :] = r0.astype(jnp.int32)
        rs1[pl.ds(base, TS), :] = r1.astype(jnp.int32)
        w0_ref[...] = w0
        w1_ref[...] = 1.0 - w0
        xp_ref[...] = _pack_bf16(x)

    @pl.when(i == nb)
    def _():
        cf = carry_ref[0:1, :]
        tiles_row = jnp.ceil(cf / TM)
        i8r = lax.broadcasted_iota(jnp.int32, (E, E), 0)
        i8c = lax.broadcasted_iota(jnp.int32, (E, E), 1)
        le_t = (i8r <= i8c).astype(jnp.float32)
        cum_row = jnp.dot(tiles_row, le_t,
                          precision=jax.lax.Precision.HIGHEST)
        off_row = (cum_row - tiles_row) * TM

        ii = lax.broadcasted_iota(jnp.int32, (T, E), 1)
        zero = jnp.zeros((), jnp.float32)
        oh0 = ii == es0[...]
        oh1 = ii == es1[...]
        p0 = jnp.sum(jnp.where(oh0, off_row, zero), axis=1, keepdims=True)
        p1 = jnp.sum(jnp.where(oh1, off_row, zero), axis=1, keepdims=True)
        p0_ref[...] = p0.astype(jnp.int32) + rs0[...]
        p1_ref[...] = p1.astype(jnp.int32) + rs1[...]

        eye = (i8r == i8c).astype(jnp.float32)
        cum_col = jnp.sum(eye * cum_row, axis=1, keepdims=True)
        it = lax.broadcasted_iota(jnp.int32, (E, NT), 1).astype(jnp.float32)
        ge = (it >= cum_col).astype(jnp.int32)
        texp = jnp.minimum(jnp.sum(ge, axis=0, keepdims=True), E - 1)
        texp_ref[...] = jnp.broadcast_to(texp, (8, NT))


def _router(x, rw):
    nb = T // TS
    meta_i = jax.ShapeDtypeStruct((T, 1), jnp.int32)
    meta_f = jax.ShapeDtypeStruct((T, 1), jnp.float32)
    blk = lambda i: (jnp.minimum(i, nb - 1), 0)
    col = lambda: pl.BlockSpec((TS, 1), blk)
    full = lambda: pl.BlockSpec((T, 1), lambda i: (0, 0))
    return pl.pallas_call(
        _router_body,
        grid=(nb + 1,),
        in_specs=[
            pl.BlockSpec((TS, D), blk),
            pl.BlockSpec((D, E), lambda i: (0, 0)),
        ],
        out_specs=[
            pl.BlockSpec((TS, DH), blk),
            col(), col(), full(), full(),
            pl.BlockSpec((8, NT), lambda i: (0, 0)),
        ],
        out_shape=[
            jax.ShapeDtypeStruct((T, DH), jnp.int32),
            meta_f, meta_f, meta_i, meta_i,
            jax.ShapeDtypeStruct((8, NT), jnp.int32),
        ],
        scratch_shapes=[pltpu.VMEM((8, E), jnp.float32),
                        pltpu.VMEM((T, 1), jnp.int32),
                        pltpu.VMEM((T, 1), jnp.int32),
                        pltpu.VMEM((T, 1), jnp.int32),
                        pltpu.VMEM((T, 1), jnp.int32)],
    )(x, rw)


def _shared_body(x_ref, sg_ref, su_ref, sd_ref, shared_ref):
    xb = x_ref[...]
    sg = jnp.dot(xb, sg_ref[...], preferred_element_type=jnp.float32)
    su = jnp.dot(xb, su_ref[...], preferred_element_type=jnp.float32)
    h = sg * jax.nn.sigmoid(sg) * su
    shared_ref[...] = jnp.dot(h, sd_ref[...], preferred_element_type=jnp.float32)


def _shared_mlp(x, sg, su, sd):
    nb = T // TS
    return pl.pallas_call(
        _shared_body,
        grid=(nb,),
        in_specs=[
            pl.BlockSpec((TS, D), lambda i: (i, 0)),
            pl.BlockSpec((D, FS), lambda i: (0, 0)),
            pl.BlockSpec((D, FS), lambda i: (0, 0)),
            pl.BlockSpec((FS, D), lambda i: (0, 0)),
        ],
        out_specs=pl.BlockSpec((TS, D), lambda i: (i, 0)),
        out_shape=jax.ShapeDtypeStruct((T, D), jnp.float32),
    )(x, sg, su, sd)


def _sc_scatter_rows(src, pos3, ch):
    t, d = src.shape
    per = t // NW
    nch = per // ch
    mesh = plsc.VectorSubcoreMesh(core_axis_name="c", subcore_axis_name="s")

    @functools.partial(
        pl.kernel, mesh=mesh,
        out_type=jax.ShapeDtypeStruct((NPAD, d), src.dtype),
        scratch_types=[
            pltpu.VMEM((nch, ch), jnp.int32),
            pltpu.VMEM((nch, ch), jnp.int32),
            [pltpu.VMEM((ch, d), src.dtype) for _ in range(2)],
            [pltpu.SemaphoreType.DMA for _ in range(2)],
            [pltpu.SemaphoreType.DMA for _ in range(2)],
            [pltpu.SemaphoreType.DMA for _ in range(2)],
        ],
    )
    def k(src_hbm, pos3_hbm, out_hbm, idx0_v, idx1_v, bufs, lsems, s0sems, s1sems):
        wid = lax.axis_index("s") * 2 + lax.axis_index("c")
        base = wid * per
        pltpu.sync_copy(pos3_hbm.at[0, pl.ds(wid * nch, nch)], idx0_v)
        pltpu.sync_copy(pos3_hbm.at[1, pl.ds(wid * nch, nch)], idx1_v)
        lops = [None] * nch
        sops = {}

        def load(j):
            b = j % 2
            lops[j] = pltpu.async_copy(
                src_hbm.at[pl.ds(base + j * ch, ch)], bufs[b], lsems[b])

        load(0)
        for j in range(nch):
            if j + 1 < nch:
                if j - 1 >= 0:
                    sops[(j - 1, 0)].wait()
                    sops[(j - 1, 1)].wait()
                load(j + 1)
            lops[j].wait()
            b = j % 2
            sops[(j, 0)] = pltpu.async_copy(
                bufs[b], out_hbm.at[idx0_v.at[j]], s0sems[b])
            sops[(j, 1)] = pltpu.async_copy(
                bufs[b], out_hbm.at[idx1_v.at[j]], s1sems[b])
        for j in range(max(0, nch - 2), nch):
            sops[(j, 0)].wait()
            sops[(j, 1)].wait()

    return k(src, pos3)


def _sc_gather_rows(src, idx, ch):
    m = idx.shape[0]
    d = src.shape[1]
    per = m // NW
    nch = per // ch
    nbuf = 2
    mesh = plsc.VectorSubcoreMesh(core_axis_name="c", subcore_axis_name="s")

    @functools.partial(
        pl.kernel, mesh=mesh,
        out_type=jax.ShapeDtypeStruct((m, d), src.dtype),
        scratch_types=[
            pltpu.VMEM((per,), jnp.int32),
            [pltpu.VMEM((ch, d), src.dtype) for _ in range(nbuf)],
            [pltpu.SemaphoreType.DMA for _ in range(nbuf)],
            [pltpu.SemaphoreType.DMA for _ in range(nbuf)],
        ],
    )
    def k(src_hbm, idx_hbm, out_hbm, idx_v, bufs, gsems, wsems):
        wid = lax.axis_index("s") * 2 + lax.axis_index("c")
        base = wid * per
        pltpu.sync_copy(idx_hbm.at[pl.ds(base, per)], idx_v)
        gops = [None] * nch
        wops = [None] * nch

        def start_gather(c):
            b = c % nbuf
            gops[c] = pltpu.async_copy(
                src_hbm.at[idx_v.at[pl.ds(c * ch, ch)]], bufs[b], gsems[b])

        start_gather(0)
        for c in range(nch):
            if c + 1 < nch:
                if c + 1 >= nbuf:
                    wops[c + 1 - nbuf].wait()
                start_gather(c + 1)
            gops[c].wait()
            b = c % nbuf
            wops[c] = pltpu.async_copy(
                bufs[b], out_hbm.at[pl.ds(base + c * ch, ch)], wsems[b])
        for c in range(max(0, nch - nbuf), nch):
            wops[c].wait()

    return k(src, idx)


def _grouped_body(texp_s, xg_ref, wg_ref, wu_ref, wd_ref, y_ref):
    xg = _unpack_bf16(xg_ref[...]).astype(jnp.float32)
    hg = jnp.dot(xg, wg_ref[0], preferred_element_type=jnp.float32)
    hu = jnp.dot(xg, wu_ref[0], preferred_element_type=jnp.float32)
    h = hg * jax.nn.sigmoid(hg) * hu
    y = jnp.dot(h, wd_ref[0], preferred_element_type=jnp.float32)
    y_ref[...] = _pack_bf16(y)


def _grouped_mlp(texp, xg, wg, wu, wd):
    grid_spec = pltpu.PrefetchScalarGridSpec(
        num_scalar_prefetch=1,
        grid=(NT,),
        in_specs=[
            pl.BlockSpec((TM, DH), lambda i, s: (i, 0)),
            pl.BlockSpec((1, D, F), lambda i, s: (s[i], 0, 0)),
            pl.BlockSpec((1, D, F), lambda i, s: (s[i], 0, 0)),
            pl.BlockSpec((1, F, D), lambda i, s: (s[i], 0, 0)),
        ],
        out_specs=pl.BlockSpec((TM, DH), lambda i, s: (i, 0)),
    )
    return pl.pallas_call(
        _grouped_body, grid_spec=grid_spec,
        out_shape=jax.ShapeDtypeStruct((NPAD, DH), jnp.int32),
    )(texp, xg, wg, wu, wd)


def _combine_body(ya_ref, yb_ref, sh_ref, w0_ref, w1_ref, o_ref):
    ya = _unpack_bf16(ya_ref[...]).astype(jnp.float32)
    yb = _unpack_bf16(yb_ref[...]).astype(jnp.float32)
    o_ref[...] = w0_ref[...] * ya + w1_ref[...] * yb + sh_ref[...]


def _combine(yg, shared_out, w0, w1):
    nb = T // TS
    return pl.pallas_call(
        _combine_body,
        grid=(nb,),
        in_specs=[
            pl.BlockSpec((TS, DH), lambda i: (i, 0)),
            pl.BlockSpec((TS, DH), lambda i, _nb=nb: (i + _nb, 0)),
            pl.BlockSpec((TS, D), lambda i: (i, 0)),
            pl.BlockSpec((TS, 1), lambda i: (i, 0)),
            pl.BlockSpec((TS, 1), lambda i: (i, 0)),
        ],
        out_specs=pl.BlockSpec((TS, D), lambda i: (i, 0)),
        out_shape=jax.ShapeDtypeStruct((T, D), jnp.float32),
    )(yg, yg, shared_out, w0, w1)


def kernel(hidden_states, router_w, w_gate, w_up, w_down,
           shared_gate, shared_up, shared_down):
    x = hidden_states
    (xp, w0, w1, p0, p1, texp8) = _router(x, router_w)
    texp = texp8[0]
    pos0 = p0[:, 0]
    pos1 = p1[:, 0]
    shared_out = _shared_mlp(x, shared_gate, shared_up, shared_down)
    ch = 32
    pos3 = jnp.stack([pos0, pos1]).reshape(2, T // ch, ch)
    xg = _sc_scatter_rows(xp, pos3, ch)
    y = _grouped_mlp(texp, xg, w_gate, w_up, w_down)
    pos01 = jnp.concatenate([pos0, pos1])
    yg = _sc_gather_rows(y, pos01, 64)
    return _combine(yg, shared_out, w0, w1)


# speedup vs baseline: 1.1296x; 1.0072x over previous
import functools

import jax
import jax.numpy as jnp
from jax import lax
from jax.experimental import pallas as pl
from jax.experimental.pallas import tpu as pltpu
from jax.experimental.pallas import tpu_sc as plsc

T, D, E, K, F, FS = 4096, 1024, 8, 2, 512, 1024
TS = 1024
TM = 512
_NRAW = T * K + E * TM
NT = -(-_NRAW // TM)
NPAD = NT * TM
NW = 32
NEG = -1e30
DH = D // 2


def _pack_bf16(v):
    vb = v.astype(jnp.bfloat16)
    lo = jax.lax.bitcast_convert_type(vb[:, :DH], jnp.uint16).astype(jnp.uint32)
    hi = jax.lax.bitcast_convert_type(vb[:, DH:], jnp.uint16).astype(jnp.uint32)
    return jax.lax.bitcast_convert_type(lo | (hi << 16), jnp.int32)


def _unpack_bf16(w):
    u = jax.lax.bitcast_convert_type(w, jnp.uint32)
    lo = jax.lax.bitcast_convert_type((u & 0xFFFF).astype(jnp.uint16),
                                      jnp.bfloat16)
    hi = jax.lax.bitcast_convert_type((u >> 16).astype(jnp.uint16),
                                      jnp.bfloat16)
    return jnp.concatenate([lo, hi], axis=1)


def _router_body(x_ref, rw_ref,
                 xp_ref, w0_ref, w1_ref, p0_ref, p1_ref, texp_ref,
                 carry_ref, es0, es1, rs0, rs1):
    i = pl.program_id(0)
    nb = T // TS

    @pl.when(i == 0)
    def _():
        carry_ref[...] = jnp.zeros_like(carry_ref)

    @pl.when(i < nb)
    def _():
        x = x_ref[...]
        logits = jnp.dot(x, rw_ref[...], preferred_element_type=jnp.float32)
        ii = lax.broadcasted_iota(jnp.int32, (TS, E), 1)
        m0 = jnp.max(logits, axis=1, keepdims=True)
        e0 = jnp.min(jnp.where(logits == m0, ii, E), axis=1, keepdims=True)
        l2 = jnp.where(ii == e0, NEG, logits)
        m1 = jnp.max(l2, axis=1, keepdims=True)
        e1 = jnp.min(jnp.where(l2 == m1, ii, E), axis=1, keepdims=True)
        w0 = jax.nn.sigmoid(m0 - m1)

        oh0 = (ii == e0).astype(jnp.float32)
        oh1 = (ii == e1).astype(jnp.float32)
        oh = oh0 + oh1
        cum = oh
        k = 1
        while k < TS:
            cum = cum + jnp.concatenate(
                [jnp.zeros((k, E), jnp.float32), cum[:-k, :]], axis=0)
            k *= 2
        cum = cum - oh + carry_ref[0:1, :]
        r0 = jnp.sum(cum * oh0, axis=1, keepdims=True)
        r1 = jnp.sum(cum * oh1, axis=1, keepdims=True)
        carry_new = carry_ref[0:1, :] + jnp.sum(oh, axis=0, keepdims=True)
        carry_ref[...] = jnp.broadcast_to(carry_new, carry_ref.shape)

        base = i * TS
        es0[pl.ds(base, TS), :] = e0
        es1[pl.ds(base, TS), :] = e1
        rs0[pl.ds(base, TS), :] = r0.astype(jnp.int32)
        rs1[pl.ds(base, TS), :] = r1.astype(jnp.int32)
        w0_ref[...] = w0
        w1_ref[...] = 1.0 - w0
        xp_ref[...] = _pack_bf16(x)

    @pl.when(i == nb)
    def _():
        cf = carry_ref[0:1, :]
        tiles_row = jnp.ceil(cf / TM)
        i8r = lax.broadcasted_iota(jnp.int32, (E, E), 0)
        i8c = lax.broadcasted_iota(jnp.int32, (E, E), 1)
        le_t = (i8r <= i8c).astype(jnp.float32)
        cum_row = jnp.dot(tiles_row, le_t,
                          precision=jax.lax.Precision.HIGHEST)
        off_row = (cum_row - tiles_row) * TM

        ii = lax.broadcasted_iota(jnp.int32, (T, E), 1)
        zero = jnp.zeros((), jnp.float32)
        oh0 = ii == es0[...]
        oh1 = ii == es1[...]
        p0 = jnp.sum(jnp.where(oh0, off_row, zero), axis=1, keepdims=True)
        p1 = jnp.sum(jnp.where(oh1, off_row, zero), axis=1, keepdims=True)
        p0_ref[...] = p0.astype(jnp.int32) + rs0[...]
        p1_ref[...] = p1.astype(jnp.int32) + rs1[...]

        eye = (i8r == i8c).astype(jnp.float32)
        cum_col = jnp.sum(eye * cum_row, axis=1, keepdims=True)
        it = lax.broadcasted_iota(jnp.int32, (E, NT), 1).astype(jnp.float32)
        ge = (it >= cum_col).astype(jnp.int32)
        texp = jnp.minimum(jnp.sum(ge, axis=0, keepdims=True), E - 1)
        texp_ref[...] = jnp.broadcast_to(texp, (8, NT))


def _router(x, rw):
    nb = T // TS
    meta_i = jax.ShapeDtypeStruct((T, 1), jnp.int32)
    meta_f = jax.ShapeDtypeStruct((T, 1), jnp.float32)
    blk = lambda i: (jnp.minimum(i, nb - 1), 0)
    col = lambda: pl.BlockSpec((TS, 1), blk)
    full = lambda: pl.BlockSpec((T, 1), lambda i: (0, 0))
    return pl.pallas_call(
        _router_body,
        grid=(nb + 1,),
        in_specs=[
            pl.BlockSpec((TS, D), blk),
            pl.BlockSpec((D, E), lambda i: (0, 0)),
        ],
        out_specs=[
            pl.BlockSpec((TS, DH), blk),
            col(), col(), full(), full(),
            pl.BlockSpec((8, NT), lambda i: (0, 0)),
        ],
        out_shape=[
            jax.ShapeDtypeStruct((T, DH), jnp.int32),
            meta_f, meta_f, meta_i, meta_i,
            jax.ShapeDtypeStruct((8, NT), jnp.int32),
        ],
        scratch_shapes=[pltpu.VMEM((8, E), jnp.float32),
                        pltpu.VMEM((T, 1), jnp.int32),
                        pltpu.VMEM((T, 1), jnp.int32),
                        pltpu.VMEM((T, 1), jnp.int32),
                        pltpu.VMEM((T, 1), jnp.int32)],
    )(x, rw)


def _shared_body(x_ref, sg_ref, su_ref, sd_ref, shared_ref):
    xb = x_ref[...]
    sg = jnp.dot(xb, sg_ref[...], preferred_element_type=jnp.float32)
    su = jnp.dot(xb, su_ref[...], preferred_element_type=jnp.float32)
    h = sg * jax.nn.sigmoid(sg) * su
    shared_ref[...] = jnp.dot(h, sd_ref[...], preferred_element_type=jnp.float32)


def _shared_mlp(x, sg, su, sd):
    nb = T // TS
    return pl.pallas_call(
        _shared_body,
        grid=(nb,),
        in_specs=[
            pl.BlockSpec((TS, D), lambda i: (i, 0)),
            pl.BlockSpec((D, FS), lambda i: (0, 0)),
            pl.BlockSpec((D, FS), lambda i: (0, 0)),
            pl.BlockSpec((FS, D), lambda i: (0, 0)),
        ],
        out_specs=pl.BlockSpec((TS, D), lambda i: (i, 0)),
        out_shape=jax.ShapeDtypeStruct((T, D), jnp.float32),
    )(x, sg, su, sd)


def _sc_scatter_rows(src, pos3, ch):
    t, d = src.shape
    per = t // NW
    nch = per // ch
    mesh = plsc.VectorSubcoreMesh(core_axis_name="c", subcore_axis_name="s")

    @functools.partial(
        pl.kernel, mesh=mesh,
        out_type=jax.ShapeDtypeStruct((NPAD, d), src.dtype),
        scratch_types=[
            pltpu.VMEM((nch, ch), jnp.int32),
            pltpu.VMEM((nch, ch), jnp.int32),
            [pltpu.VMEM((ch, d), src.dtype) for _ in range(2)],
            [pltpu.SemaphoreType.DMA for _ in range(2)],
            [pltpu.SemaphoreType.DMA for _ in range(2)],
            [pltpu.SemaphoreType.DMA for _ in range(2)],
        ],
    )
    def k(src_hbm, pos3_hbm, out_hbm, idx0_v, idx1_v, bufs, lsems, s0sems, s1sems):
        wid = lax.axis_index("s") * 2 + lax.axis_index("c")
        base = wid * per
        pltpu.sync_copy(pos3_hbm.at[0, pl.ds(wid * nch, nch)], idx0_v)
        pltpu.sync_copy(pos3_hbm.at[1, pl.ds(wid * nch, nch)], idx1_v)
        lops = [None] * nch
        sops = {}

        def load(j):
            b = j % 2
            lops[j] = pltpu.async_copy(
                src_hbm.at[pl.ds(base + j * ch, ch)], bufs[b], lsems[b])

        load(0)
        for j in range(nch):
            if j + 1 < nch:
                if j - 1 >= 0:
                    sops[(j - 1, 0)].wait()
                    sops[(j - 1, 1)].wait()
                load(j + 1)
            lops[j].wait()
            b = j % 2
            sops[(j, 0)] = pltpu.async_copy(
                bufs[b], out_hbm.at[idx0_v.at[j]], s0sems[b])
            sops[(j, 1)] = pltpu.async_copy(
                bufs[b], out_hbm.at[idx1_v.at[j]], s1sems[b])
        for j in range(max(0, nch - 2), nch):
            sops[(j, 0)].wait()
            sops[(j, 1)].wait()

    return k(src, pos3)


def _sc_gather_rows(src, idx, ch):
    m = idx.shape[0]
    d = src.shape[1]
    per = m // NW
    nch = per // ch
    nbuf = 2
    mesh = plsc.VectorSubcoreMesh(core_axis_name="c", subcore_axis_name="s")

    @functools.partial(
        pl.kernel, mesh=mesh,
        out_type=jax.ShapeDtypeStruct((m, d), src.dtype),
        scratch_types=[
            pltpu.VMEM((per,), jnp.int32),
            [pltpu.VMEM((ch, d), src.dtype) for _ in range(nbuf)],
            [pltpu.SemaphoreType.DMA for _ in range(nbuf)],
            [pltpu.SemaphoreType.DMA for _ in range(nbuf)],
        ],
    )
    def k(src_hbm, idx_hbm, out_hbm, idx_v, bufs, gsems, wsems):
        wid = lax.axis_index("s") * 2 + lax.axis_index("c")
        base = wid * per
        pltpu.sync_copy(idx_hbm.at[pl.ds(base, per)], idx_v)
        gops = [None] * nch
        wops = [None] * nch

        def start_gather(c):
            b = c % nbuf
            gops[c] = pltpu.async_copy(
                src_hbm.at[idx_v.at[pl.ds(c * ch, ch)]], bufs[b], gsems[b])

        start_gather(0)
        for c in range(nch):
            if c + 1 < nch:
                if c + 1 >= nbuf:
                    wops[c + 1 - nbuf].wait()
                start_gather(c + 1)
            gops[c].wait()
            b = c % nbuf
            wops[c] = pltpu.async_copy(
                bufs[b], out_hbm.at[pl.ds(base + c * ch, ch)], wsems[b])
        for c in range(max(0, nch - nbuf), nch):
            wops[c].wait()

    return k(src, idx)


def _grouped_body(texp_s, xg_ref, wg_ref, wu_ref, wd_ref, y_ref):
    xg = _unpack_bf16(xg_ref[...]).astype(jnp.float32)
    hg = jnp.dot(xg, wg_ref[0], preferred_element_type=jnp.float32)
    hu = jnp.dot(xg, wu_ref[0], preferred_element_type=jnp.float32)
    h = hg * jax.nn.sigmoid(hg) * hu
    y = jnp.dot(h, wd_ref[0], preferred_element_type=jnp.float32)
    y_ref[...] = _pack_bf16(y)


def _grouped_mlp(texp, xg, wg, wu, wd):
    grid_spec = pltpu.PrefetchScalarGridSpec(
        num_scalar_prefetch=1,
        grid=(NT,),
        in_specs=[
            pl.BlockSpec((TM, DH), lambda i, s: (i, 0)),
            pl.BlockSpec((1, D, F), lambda i, s: (s[i], 0, 0)),
            pl.BlockSpec((1, D, F), lambda i, s: (s[i], 0, 0)),
            pl.BlockSpec((1, F, D), lambda i, s: (s[i], 0, 0)),
        ],
        out_specs=pl.BlockSpec((TM, DH), lambda i, s: (i, 0)),
    )
    return pl.pallas_call(
        _grouped_body, grid_spec=grid_spec,
        out_shape=jax.ShapeDtypeStruct((NPAD, DH), jnp.int32),
    )(texp, xg, wg, wu, wd)


def _combine_body(ya_ref, yb_ref, sh_ref, w0_ref, w1_ref, o_ref):
    ya = _unpack_bf16(ya_ref[...]).astype(jnp.float32)
    yb = _unpack_bf16(yb_ref[...]).astype(jnp.float32)
    o_ref[...] = w0_ref[...] * ya + w1_ref[...] * yb + sh_ref[...]


def _combine(yg, shared_out, w0, w1):
    nb = T // TS
    return pl.pallas_call(
        _combine_body,
        grid=(nb,),
        in_specs=[
            pl.BlockSpec((TS, DH), lambda i: (i, 0)),
            pl.BlockSpec((TS, DH), lambda i, _nb=nb: (i + _nb, 0)),
            pl.BlockSpec((TS, D), lambda i: (i, 0)),
            pl.BlockSpec((TS, 1), lambda i: (i, 0)),
            pl.BlockSpec((TS, 1), lambda i: (i, 0)),
        ],
        out_specs=pl.BlockSpec((TS, D), lambda i: (i, 0)),
        out_shape=jax.ShapeDtypeStruct((T, D), jnp.float32),
    )(yg, yg, shared_out, w0, w1)


def kernel(hidden_states, router_w, w_gate, w_up, w_down,
           shared_gate, shared_up, shared_down):
    x = hidden_states
    (xp, w0, w1, p0, p1, texp8) = _router(x, router_w)
    texp = texp8[0]
    pos0 = p0[:, 0]
    pos1 = p1[:, 0]
    shared_out = _shared_mlp(x, shared_gate, shared_up, shared_down)
    ch = 64
    pos3 = jnp.stack([pos0, pos1]).reshape(2, T // ch, ch)
    xg = _sc_scatter_rows(xp, pos3, ch)
    y = _grouped_mlp(texp, xg, w_gate, w_up, w_down)
    pos01 = jnp.concatenate([pos0, pos1])
    yg = _sc_gather_rows(y, pos01, 64)
    return _combine(yg, shared_out, w0, w1)
